# baseline (speedup 1.0000x reference)
import functools

import jax
import jax.numpy as jnp
from jax import lax
from jax.experimental import pallas as pl
from jax.experimental.pallas import tpu as pltpu
from jax.experimental.pallas import tpu_sc as plsc

N_NODES = 10000
N_EDGES = 320000
FDIM = 128
NB = 2
NT = 4

NTILES = 16
CHUNK = 128
IBLK = 16
CPT = 160
NBLK = CPT // IBLK
EPAD = NTILES * CPT * CHUNK
NP = 10240
ROWS_PT = NP // NTILES
RB = 2048


def _make_spmv(ngroups: int, width: int = FDIM, nprow_u: int = NP):
    mesh = plsc.VectorSubcoreMesh(core_axis_name="c", subcore_axis_name="s",
                                  num_cores=NB, num_subcores=NTILES)

    def body(u_hbm, src_hbm, dst_hbm, zeros_hbm, out_hbm,
             sidx, didx, rows, acc, sems, sem_i):
        c = lax.axis_index("c")
        s = lax.axis_index("s")
        row0 = s * ROWS_PT
        base = s * CPT

        def idx_src(bk, bp):
            return (src_hbm.at[pl.ds(base + bk * IBLK, IBLK), :], sidx.at[bp],
                    dst_hbm.at[pl.ds(base + bk * IBLK, IBLK), :], didx.at[bp])

        for g in range(ngroups):
            ug = u_hbm.at[c, g]
            pltpu.sync_copy(zeros_hbm, acc.at[pl.ds(row0, ROWS_PT), :])
            plsc.subcore_barrier()

            ss, sd, ds_, dd = idx_src(0, 0)
            pltpu.sync_copy(ss, sd)
            pltpu.sync_copy(ds_, dd)

            def block(bk, _):
                bp = lax.rem(bk, 2)

                @pl.when(bk + 1 < NBLK)
                def _():
                    ss, sd, ds_, dd = idx_src(bk + 1, 1 - bp)
                    pltpu.async_copy(ss, sd, sem_i.at[1 - bp])
                    pltpu.async_copy(ds_, dd, sem_i.at[1 - bp])

                pltpu.async_copy(ug.at[sidx.at[bp, 0]], rows.at[0], sems.at[0])

                def chunk(i, _):
                    p = lax.rem(i, 2)

                    @pl.when(i + 1 < IBLK)
                    def _():
                        pltpu.async_copy(ug.at[sidx.at[bp, i + 1]],
                                         rows.at[1 - p], sems.at[1 - p])

                    pltpu.make_async_copy(ug.at[sidx.at[bp, i]], rows.at[p],
                                          sems.at[p]).wait()
                    pltpu.sync_copy(rows.at[p], acc.at[didx.at[bp, i]],
                                    add=True)
                    return ()

                lax.fori_loop(0, IBLK, chunk, (), unroll=False)

                @pl.when(bk + 1 < NBLK)
                def _():
                    ss, sd, ds_, dd = idx_src(bk + 1, 1 - bp)
                    pltpu.make_async_copy(ss, sd, sem_i.at[1 - bp]).wait()
                    pltpu.make_async_copy(ds_, dd, sem_i.at[1 - bp]).wait()
                return ()

            lax.fori_loop(0, NBLK, block, (), unroll=False)
            plsc.subcore_barrier()
            pltpu.sync_copy(acc.at[pl.ds(row0, ROWS_PT), :],
                            out_hbm.at[c, g, pl.ds(row0, ROWS_PT), :])

    return pl.kernel(
        body,
        out_type=jax.ShapeDtypeStruct((NB, ngroups, NP, width), jnp.float32),
        mesh=mesh,
        scratch_types=[
            pltpu.VMEM((2, IBLK, CHUNK), jnp.int32),
            pltpu.VMEM((2, IBLK, CHUNK), jnp.int32),
            pltpu.VMEM((2, CHUNK, width), jnp.float32),
            pltpu.VMEM_SHARED((NP, width), jnp.float32),
            pltpu.SemaphoreType.DMA((2,)),
            pltpu.SemaphoreType.DMA((2,)),
        ],
    )


def _dinv_kernel(deg_ref, o_ref):
    d = deg_ref[:, 0:1]
    dv = jnp.where(d > 0.0, lax.rsqrt(jnp.maximum(d, 1e-30)), 0.0)
    o_ref[...] = jnp.broadcast_to(dv, o_ref.shape)


def _prep_kernel(x_ref, dinv_ref, o_ref):
    o_ref[...] = x_ref[...] * dinv_ref[...]


def _stepA_kernel(x_ref, sx_ref, h_ref, sh_ref, dinv_ref,
                  wxz_ref, whz_ref, wxr_ref, whr_ref, wxh_ref,
                  bz_ref, br_ref, bxh_ref,
                  z_ref, g_ref, gu_ref, p_ref):
    dinv = dinv_ref[...]
    x = x_ref[0]
    h = h_ref[0]
    sx = -dinv * sx_ref[0]
    sh = -dinv * sh_ref[0]

    def mm(a, w):
        return jnp.dot(a, w, preferred_element_type=jnp.float32)

    zpre = (mm(x, wxz_ref[0]) + mm(sx, wxz_ref[1])
            + mm(h, whz_ref[0]) + mm(sh, whz_ref[1]) + bz_ref[...])
    rpre = (mm(x, wxr_ref[0]) + mm(sx, wxr_ref[1])
            + mm(h, whr_ref[0]) + mm(sh, whr_ref[1]) + br_ref[...])
    z = jax.nn.sigmoid(zpre)
    r = jax.nn.sigmoid(rpre)
    g = h * r
    z_ref[0] = z
    g_ref[0] = g
    gu_ref[0] = dinv * g
    p_ref[0] = mm(x, wxh_ref[0]) + mm(sx, wxh_ref[1]) + bxh_ref[...]


def _stepB_kernel(p_ref, g_ref, sg_ref, z_ref, h_ref, dinv_ref, whh_ref,
                  bhh_ref, hn_ref, hu_ref):
    dinv = dinv_ref[...]
    sg = -dinv * sg_ref[0]

    def mm(a, w):
        return jnp.dot(a, w, preferred_element_type=jnp.float32)

    ht = jnp.tanh(p_ref[0] + mm(g_ref[0], whh_ref[0])
                  + mm(sg, whh_ref[1]) + bhh_ref[...])
    z = z_ref[0]
    hn = z * h_ref[0] + (1.0 - z) * ht
    hn_ref[0] = hn
    hu_ref[0] = dinv * hn


def _heads_kernel(h_ref, w4_ref, b4_ref, ms_ref, mix_ref, acc_ref):
    r = pl.program_id(1)
    nr = pl.num_programs(1)
    h = h_ref[0]
    pre = jnp.dot(h, w4_ref[...], preferred_element_type=jnp.float32) + b4_ref[...]
    mu = jax.nn.sigmoid(pre[:, 0:2])
    sg = jax.nn.softplus(pre[:, 2:4])
    ms_ref[0] = jnp.concatenate([mu, sg, pre[:, 4:8]], axis=1)

    base = r * RB
    ridx = base + lax.broadcasted_iota(jnp.int32, (RB, 1), 0)
    masked = jnp.where(ridx < N_NODES, h, 0.0)

    @pl.when(r == 0)
    def _():
        acc_ref[...] = jnp.zeros_like(acc_ref)

    acc_ref[...] += jnp.sum(masked, axis=0, keepdims=True)

    @pl.when(r == nr - 1)
    def _():
        m = acc_ref[...] / float(N_NODES)
        m = m - jnp.max(m, axis=1, keepdims=True)
        e = jnp.exp(m)
        mix_ref[0] = e / jnp.sum(e, axis=1, keepdims=True)


def _tc_full(shape):
    return pl.BlockSpec(shape, lambda *a: tuple(0 for _ in shape))


def kernel(in_tensor, edge_index, W_xz, b_xz, W_hz, b_hz, W_xr, b_xr,
           W_hr, b_hr, W_xh, b_xh, W_hh, b_hh, W_mu, b_mu, W_sig, b_sig):
    f32 = jnp.float32
    nbt = NB * NT
    nr = NP // RB

    src = edge_index[0]
    dst = edge_index[1]
    pad = jnp.full((EPAD - N_EDGES,), N_NODES, jnp.int32)
    src_p = jnp.concatenate([src, pad]).reshape(EPAD // CHUNK, CHUNK)
    dst_p = jnp.concatenate([dst, pad]).reshape(EPAD // CHUNK, CHUNK)
    zeros_blk = jnp.zeros((ROWS_PT, FDIM), f32)

    x_pad = jnp.zeros((NB, NT, NP, FDIM), f32)
    x_pad = x_pad.at[:, :, :N_NODES, :].set(in_tensor)

    spmv1 = _make_spmv(1)
    ones_u = jnp.ones((NB, 1, NP, FDIM), f32)
    deg_out = spmv1(ones_u, src_p, src_p, zeros_blk)
    deg = deg_out[0, 0]

    dinv = pl.pallas_call(
        _dinv_kernel,
        grid=(nr,),
        in_specs=[pl.BlockSpec((RB, FDIM), lambda r: (r, 0))],
        out_specs=pl.BlockSpec((RB, FDIM), lambda r: (r, 0)),
        out_shape=jax.ShapeDtypeStruct((NP, FDIM), f32),
    )(deg)

    x_flat = x_pad.reshape(nbt, NP, FDIM)
    u_x = pl.pallas_call(
        _prep_kernel,
        grid=(nbt, nr),
        in_specs=[
            pl.BlockSpec((1, RB, FDIM), lambda i, r: (i, r, 0)),
            pl.BlockSpec((RB, FDIM), lambda i, r: (r, 0)),
        ],
        out_specs=pl.BlockSpec((1, RB, FDIM), lambda i, r: (i, r, 0)),
        out_shape=jax.ShapeDtypeStruct((nbt, NP, FDIM), f32),
    )(x_flat, dinv)

    u_x4 = u_x.reshape(NB, NT, NP, FDIM)
    sx_list = [spmv1(u_x4[:, t:t + 1], src_p, dst_p, zeros_blk)
               for t in range(NT)]
    sx_all = jnp.concatenate(sx_list, axis=1)

    wb = _tc_full((2, FDIM, FDIM))
    bb = _tc_full((1, FDIM))
    nf = pl.BlockSpec((1, RB, FDIM), lambda b, r: (b, r, 0))
    df = pl.BlockSpec((RB, FDIM), lambda b, r: (r, 0))

    b_z = (b_xz + b_hz).reshape(1, FDIM)
    b_r = (b_xr + b_hr).reshape(1, FDIM)
    b_xh2 = b_xh.reshape(1, FDIM)
    b_hh2 = b_hh.reshape(1, FDIM)

    stepA = pl.pallas_call(
        _stepA_kernel,
        grid=(NB, nr),
        in_specs=[nf, nf, nf, nf, df, wb, wb, wb, wb, wb, bb, bb, bb],
        out_specs=[nf, nf, nf, nf],
        out_shape=[jax.ShapeDtypeStruct((NB, NP, FDIM), f32)] * 4,
    )
    stepB = pl.pallas_call(
        _stepB_kernel,
        grid=(NB, nr),
        in_specs=[nf, nf, nf, nf, nf, df, wb, bb],
        out_specs=[nf, nf],
        out_shape=[jax.ShapeDtypeStruct((NB, NP, FDIM), f32)] * 2,
    )

    zeros_nf = jnp.zeros((NB, NP, FDIM), f32)
    h = zeros_nf
    hu = None
    for t in range(NT):
        x_t = x_pad[:, t]
        sx_t = sx_all[:, t]
        if t == 0:
            sh_raw = zeros_nf
        else:
            sh_raw = spmv1(hu[:, None], src_p, dst_p, zeros_blk)[:, 0]
        z, g, gu, p = stepA(x_t, sx_t, h, sh_raw, dinv,
                            W_xz, W_hz, W_xr, W_hr, W_xh,
                            b_z, b_r, b_xh2)
        if t == 0:
            sg_raw = zeros_nf
        else:
            sg_raw = spmv1(gu[:, None], src_p, dst_p, zeros_blk)[:, 0]
        h, hu = stepB(p, g, sg_raw, z, h, dinv, W_hh, b_hh2)

    w4 = jnp.zeros((FDIM, 8), f32)
    w4 = w4.at[:, 0:2].set(W_mu).at[:, 2:4].set(W_sig)
    b4 = jnp.zeros((1, 8), f32)
    b4 = b4.at[0, 0:2].set(b_mu).at[0, 2:4].set(b_sig)

    ms, mix = pl.pallas_call(
        _heads_kernel,
        grid=(NB, nr),
        in_specs=[nf, _tc_full((FDIM, 8)), _tc_full((1, 8))],
        out_specs=[
            pl.BlockSpec((1, RB, 8), lambda b, r: (b, r, 0)),
            pl.BlockSpec((1, 1, FDIM), lambda b, r: (b, 0, 0)),
        ],
        out_shape=[
            jax.ShapeDtypeStruct((NB, NP, 8), f32),
            jax.ShapeDtypeStruct((NB, 1, FDIM), f32),
        ],
        scratch_shapes=[pltpu.VMEM((1, FDIM), f32)],
    )(h, w4, b4)

    mu = ms[:, :N_NODES, 0:2]
    sigma = ms[:, :N_NODES, 2:4]
    return (mu, sigma, mix[:, 0, :])


# speedup vs baseline: 1.0659x; 1.0659x over previous
import functools

import jax
import jax.numpy as jnp
from jax import lax
from jax.experimental import pallas as pl
from jax.experimental.pallas import tpu as pltpu
from jax.experimental.pallas import tpu_sc as plsc

N_NODES = 10000
N_EDGES = 320000
FDIM = 128
NB = 2
NT = 4

NTILES = 16
CHUNK = 128
IBLK = 16
CPT = 160
NBLK = CPT // IBLK
EPAD = NTILES * CPT * CHUNK
NP = 10240
ROWS_PT = NP // NTILES
RB = 2048


def _make_spmv(ngroups: int, width: int = FDIM, nprow_u: int = NP):
    mesh = plsc.VectorSubcoreMesh(core_axis_name="c", subcore_axis_name="s",
                                  num_cores=NB, num_subcores=NTILES)

    def body(u_hbm, src_hbm, dst_hbm, zeros_hbm, out_hbm,
             sidx, didx, rows, acc, sems, sem_i):
        c = lax.axis_index("c")
        s = lax.axis_index("s")
        row0 = s * ROWS_PT
        base = s * CPT

        def idx_src(bk, bp):
            return (src_hbm.at[pl.ds(base + bk * IBLK, IBLK), :], sidx.at[bp],
                    dst_hbm.at[pl.ds(base + bk * IBLK, IBLK), :], didx.at[bp])

        for g in range(ngroups):
            ug = u_hbm.at[c, g]
            pltpu.sync_copy(zeros_hbm, acc.at[pl.ds(row0, ROWS_PT), :])
            plsc.subcore_barrier()

            ss, sd, ds_, dd = idx_src(0, 0)
            pltpu.sync_copy(ss, sd)
            pltpu.sync_copy(ds_, dd)

            def block(bk, _):
                bp = lax.rem(bk, 2)

                @pl.when(bk + 1 < NBLK)
                def _():
                    ss, sd, ds_, dd = idx_src(bk + 1, 1 - bp)
                    pltpu.async_copy(ss, sd, sem_i.at[1 - bp])
                    pltpu.async_copy(ds_, dd, sem_i.at[1 - bp])

                pltpu.async_copy(ug.at[sidx.at[bp, 0]], rows.at[0], sems.at[0])

                def chunk(i, _):
                    p = lax.rem(i, 2)

                    @pl.when(i + 1 < IBLK)
                    def _():
                        pltpu.async_copy(ug.at[sidx.at[bp, i + 1]],
                                         rows.at[1 - p], sems.at[1 - p])

                    pltpu.make_async_copy(ug.at[sidx.at[bp, i]], rows.at[p],
                                          sems.at[p]).wait()
                    pltpu.sync_copy(rows.at[p], acc.at[didx.at[bp, i]],
                                    add=True)
                    return ()

                lax.fori_loop(0, IBLK, chunk, (), unroll=False)

                @pl.when(bk + 1 < NBLK)
                def _():
                    ss, sd, ds_, dd = idx_src(bk + 1, 1 - bp)
                    pltpu.make_async_copy(ss, sd, sem_i.at[1 - bp]).wait()
                    pltpu.make_async_copy(ds_, dd, sem_i.at[1 - bp]).wait()
                return ()

            lax.fori_loop(0, NBLK, block, (), unroll=False)
            plsc.subcore_barrier()
            pltpu.sync_copy(acc.at[pl.ds(row0, ROWS_PT), :],
                            out_hbm.at[c, g, pl.ds(row0, ROWS_PT), :])

    return pl.kernel(
        body,
        out_type=jax.ShapeDtypeStruct((NB, ngroups, NP, width), jnp.float32),
        mesh=mesh,
        scratch_types=[
            pltpu.VMEM((2, IBLK, CHUNK), jnp.int32),
            pltpu.VMEM((2, IBLK, CHUNK), jnp.int32),
            pltpu.VMEM((2, CHUNK, width), jnp.float32),
            pltpu.VMEM_SHARED((NP, width), jnp.float32),
            pltpu.SemaphoreType.DMA((2,)),
            pltpu.SemaphoreType.DMA((2,)),
        ],
    )


def _dinv_kernel(deg_ref, o_ref):
    d = deg_ref[:, 0:1]
    dv = jnp.where(d > 0.0, lax.rsqrt(jnp.maximum(d, 1e-30)), 0.0)
    o_ref[...] = jnp.broadcast_to(dv, o_ref.shape)


def _prep_kernel(x_ref, dinv_ref, o_ref):
    o_ref[...] = x_ref[...] * dinv_ref[...]


def _stepA_kernel(x_ref, sx_ref, h_ref, sh_ref, dinv_ref,
                  wxz_ref, whz_ref, wxr_ref, whr_ref, wxh_ref,
                  bz_ref, br_ref, bxh_ref,
                  z_ref, g_ref, gu_ref, p_ref):
    dinv = dinv_ref[...]
    x = x_ref[0]
    h = h_ref[0]
    sx = -dinv * sx_ref[0]
    sh = -dinv * sh_ref[0]

    def mm(a, w):
        return jnp.dot(a, w, preferred_element_type=jnp.float32)

    zpre = (mm(x, wxz_ref[0]) + mm(sx, wxz_ref[1])
            + mm(h, whz_ref[0]) + mm(sh, whz_ref[1]) + bz_ref[...])
    rpre = (mm(x, wxr_ref[0]) + mm(sx, wxr_ref[1])
            + mm(h, whr_ref[0]) + mm(sh, whr_ref[1]) + br_ref[...])
    z = jax.nn.sigmoid(zpre)
    r = jax.nn.sigmoid(rpre)
    g = h * r
    z_ref[0] = z
    g_ref[0] = g
    gu_ref[0] = dinv * g
    p_ref[0] = mm(x, wxh_ref[0]) + mm(sx, wxh_ref[1]) + bxh_ref[...]


def _stepB_kernel(p_ref, g_ref, sg_ref, z_ref, h_ref, dinv_ref, whh_ref,
                  bhh_ref, hn_ref, hu_ref):
    dinv = dinv_ref[...]
    sg = -dinv * sg_ref[0]

    def mm(a, w):
        return jnp.dot(a, w, preferred_element_type=jnp.float32)

    ht = jnp.tanh(p_ref[0] + mm(g_ref[0], whh_ref[0])
                  + mm(sg, whh_ref[1]) + bhh_ref[...])
    z = z_ref[0]
    hn = z * h_ref[0] + (1.0 - z) * ht
    hn_ref[0] = hn
    hu_ref[0] = dinv * hn


def _heads_kernel(h_ref, w4_ref, b4_ref, ms_ref, mix_ref, acc_ref):
    r = pl.program_id(1)
    nr = pl.num_programs(1)
    h = h_ref[0]
    pre = jnp.dot(h, w4_ref[...], preferred_element_type=jnp.float32) + b4_ref[...]
    mu = jax.nn.sigmoid(pre[:, 0:2])
    sg = jax.nn.softplus(pre[:, 2:4])
    ms_ref[0] = jnp.concatenate([mu, sg, pre[:, 4:8]], axis=1)

    base = r * RB
    ridx = base + lax.broadcasted_iota(jnp.int32, (RB, 1), 0)
    masked = jnp.where(ridx < N_NODES, h, 0.0)

    @pl.when(r == 0)
    def _():
        acc_ref[...] = jnp.zeros_like(acc_ref)

    acc_ref[...] += jnp.sum(masked, axis=0, keepdims=True)

    @pl.when(r == nr - 1)
    def _():
        m = acc_ref[...] / float(N_NODES)
        m = m - jnp.max(m, axis=1, keepdims=True)
        e = jnp.exp(m)
        mix_ref[0] = e / jnp.sum(e, axis=1, keepdims=True)


def _tc_full(shape):
    return pl.BlockSpec(shape, lambda *a: tuple(0 for _ in shape))


def kernel(in_tensor, edge_index, W_xz, b_xz, W_hz, b_hz, W_xr, b_xr,
           W_hr, b_hr, W_xh, b_xh, W_hh, b_hh, W_mu, b_mu, W_sig, b_sig):
    f32 = jnp.float32
    nbt = NB * NT
    nr = NP // RB

    src = edge_index[0]
    dst = edge_index[1]
    pad = jnp.full((EPAD - N_EDGES,), N_NODES, jnp.int32)
    src_p = jnp.concatenate([src, pad]).reshape(EPAD // CHUNK, CHUNK)
    dst_p = jnp.concatenate([dst, pad]).reshape(EPAD // CHUNK, CHUNK)
    zeros_blk = jnp.zeros((ROWS_PT, FDIM), f32)

    x_pad = jnp.zeros((NB, NT, NP, FDIM), f32)
    x_pad = x_pad.at[:, :, :N_NODES, :].set(in_tensor)

    spmv1 = _make_spmv(1)
    spmv3 = _make_spmv(NT - 1)
    ones_u = jnp.ones((NB, 1, NP, FDIM), f32)
    deg_out = spmv1(ones_u, src_p, src_p, zeros_blk)
    deg = deg_out[0, 0]

    dinv = pl.pallas_call(
        _dinv_kernel,
        grid=(nr,),
        in_specs=[pl.BlockSpec((RB, FDIM), lambda r: (r, 0))],
        out_specs=pl.BlockSpec((RB, FDIM), lambda r: (r, 0)),
        out_shape=jax.ShapeDtypeStruct((NP, FDIM), f32),
    )(deg)

    x_flat = x_pad.reshape(nbt, NP, FDIM)
    u_x = pl.pallas_call(
        _prep_kernel,
        grid=(nbt, nr),
        in_specs=[
            pl.BlockSpec((1, RB, FDIM), lambda i, r: (i, r, 0)),
            pl.BlockSpec((RB, FDIM), lambda i, r: (r, 0)),
        ],
        out_specs=pl.BlockSpec((1, RB, FDIM), lambda i, r: (i, r, 0)),
        out_shape=jax.ShapeDtypeStruct((nbt, NP, FDIM), f32),
    )(x_flat, dinv)

    u_x4 = u_x.reshape(NB, NT, NP, FDIM)
    sx0 = spmv1(u_x4[:, 0:1], src_p, dst_p, zeros_blk)
    sx123 = spmv3(u_x4[:, 1:], src_p, dst_p, zeros_blk)

    wb = _tc_full((2, FDIM, FDIM))
    bb = _tc_full((1, FDIM))
    nf = pl.BlockSpec((1, RB, FDIM), lambda b, r: (b, r, 0))
    df = pl.BlockSpec((RB, FDIM), lambda b, r: (r, 0))

    b_z = (b_xz + b_hz).reshape(1, FDIM)
    b_r = (b_xr + b_hr).reshape(1, FDIM)
    b_xh2 = b_xh.reshape(1, FDIM)
    b_hh2 = b_hh.reshape(1, FDIM)

    stepA = pl.pallas_call(
        _stepA_kernel,
        grid=(NB, nr),
        in_specs=[nf, nf, nf, nf, df, wb, wb, wb, wb, wb, bb, bb, bb],
        out_specs=[nf, nf, nf, nf],
        out_shape=[jax.ShapeDtypeStruct((NB, NP, FDIM), f32)] * 4,
    )
    stepB = pl.pallas_call(
        _stepB_kernel,
        grid=(NB, nr),
        in_specs=[nf, nf, nf, nf, nf, df, wb, bb],
        out_specs=[nf, nf],
        out_shape=[jax.ShapeDtypeStruct((NB, NP, FDIM), f32)] * 2,
    )

    zeros_nf = jnp.zeros((NB, NP, FDIM), f32)
    h = zeros_nf
    hu = None
    for t in range(NT):
        x_t = x_pad[:, t]
        sx_t = sx0[:, 0] if t == 0 else sx123[:, t - 1]
        if t == 0:
            sh_raw = zeros_nf
        else:
            sh_raw = spmv1(hu[:, None], src_p, dst_p, zeros_blk)[:, 0]
        z, g, gu, p = stepA(x_t, sx_t, h, sh_raw, dinv,
                            W_xz, W_hz, W_xr, W_hr, W_xh,
                            b_z, b_r, b_xh2)
        if t == 0:
            sg_raw = zeros_nf
        else:
            sg_raw = spmv1(gu[:, None], src_p, dst_p, zeros_blk)[:, 0]
        h, hu = stepB(p, g, sg_raw, z, h, dinv, W_hh, b_hh2)

    w4 = jnp.zeros((FDIM, 8), f32)
    w4 = w4.at[:, 0:2].set(W_mu).at[:, 2:4].set(W_sig)
    b4 = jnp.zeros((1, 8), f32)
    b4 = b4.at[0, 0:2].set(b_mu).at[0, 2:4].set(b_sig)

    ms, mix = pl.pallas_call(
        _heads_kernel,
        grid=(NB, nr),
        in_specs=[nf, _tc_full((FDIM, 8)), _tc_full((1, 8))],
        out_specs=[
            pl.BlockSpec((1, RB, 8), lambda b, r: (b, r, 0)),
            pl.BlockSpec((1, 1, FDIM), lambda b, r: (b, 0, 0)),
        ],
        out_shape=[
            jax.ShapeDtypeStruct((NB, NP, 8), f32),
            jax.ShapeDtypeStruct((NB, 1, FDIM), f32),
        ],
        scratch_shapes=[pltpu.VMEM((1, FDIM), f32)],
    )(h, w4, b4)

    mu = ms[:, :N_NODES, 0:2]
    sigma = ms[:, :N_NODES, 2:4]
    return (mu, sigma, mix[:, 0, :])
